# baseline (speedup 1.0000x reference)
import dataclasses
import functools

import jax
import jax.numpy as jnp
from jax import lax
from jax.experimental import pallas as pl
from jax.experimental.pallas import tpu as pltpu
from jax.experimental.pallas import tpu_sc as plsc

_NUM_SC_CORES = 2
_NUM_SC_SUBCORES = 16
_SC_LANES = 16


def _sc_present(flat_idx, B, V, L):
    NW = _NUM_SC_CORES * _NUM_SC_SUBCORES
    RP = B // NW
    CH = RP * V
    NI = RP * L
    mesh = plsc.VectorSubcoreMesh(core_axis_name="c", subcore_axis_name="s")
    cp = pltpu.CompilerParams()
    if "needs_layout_passes" in pltpu.CompilerParams.__dataclass_fields__:
        cp = dataclasses.replace(cp, needs_layout_passes=False)

    @functools.partial(
        pl.kernel,
        out_type=jax.ShapeDtypeStruct((B * V,), jnp.float32),
        mesh=mesh,
        compiler_params=cp,
        scratch_types=[
            pltpu.VMEM((CH,), jnp.float32),
            pltpu.VMEM((NI,), jnp.int32),
        ],
    )
    def k(idx_hbm, out_hbm, buf, idxv):
        wid = lax.axis_index("s") * _NUM_SC_CORES + lax.axis_index("c")
        zeros = jnp.zeros((_SC_LANES,), jnp.float32)
        ones = jnp.ones((_SC_LANES,), jnp.float32)

        pltpu.sync_copy(idx_hbm.at[pl.ds(wid * NI, NI)], idxv)

        @pl.loop(0, CH, step=_SC_LANES)
        def _(j):
            buf[pl.ds(j, _SC_LANES)] = zeros

        @pl.loop(0, NI, step=_SC_LANES)
        def _(j):
            plsc.store_scatter(buf, [idxv[pl.ds(j, _SC_LANES)]], ones)

        pltpu.sync_copy(buf, out_hbm.at[pl.ds(wid * CH, CH)])

    return k(flat_idx)


def _tc_dense(sp, sl, eosmask, gmat, BB):
    B, L, V = sl.shape
    VN = sp.shape[1]

    def body(sp_ref, sl_ref, em_ref, g_ref, ent_ref, tsq_ref, sle_ref):
        p0 = sp_ref[...]
        t = (p0 * jnp.log(p0)).astype(jnp.bfloat16)
        ent_ref[...] = -lax.dot_general(
            t, g_ref[...], (((1,), (0,)), ((), ())),
            preferred_element_type=jnp.float32)
        x = sl_ref[...]
        mn = x.min(axis=2, keepdims=True)
        xs = jnp.where(em_ref[...] != 0, mn, x)
        m = xs.max(axis=1)
        mx = m.max(axis=1, keepdims=True)
        z = m - mx
        e = jnp.exp(z)
        s = e.sum(axis=1, keepdims=True)
        p = e / s
        logp = z - jnp.log(s)
        sle_ref[...] = -(p * logp).sum(axis=1, keepdims=True)
        t = 1.0 - p
        tsq_ref[...] = (t * t).astype(jnp.bfloat16)

    return pl.pallas_call(
        body,
        grid=(B // BB,),
        in_specs=[
            pl.BlockSpec((BB, VN), lambda i: (i, 0)),
            pl.BlockSpec((BB, L, V), lambda i: (i, 0, 0)),
            pl.BlockSpec((1, 1, V), lambda i: (0, 0, 0)),
            pl.BlockSpec((VN, V), lambda i: (0, 0)),
        ],
        out_specs=[
            pl.BlockSpec((BB, V), lambda i: (i, 0)),
            pl.BlockSpec((BB, V), lambda i: (i, 0)),
            pl.BlockSpec((BB, 1), lambda i: (i, 0)),
        ],
        out_shape=[
            jax.ShapeDtypeStruct((B, V), jnp.float32),
            jax.ShapeDtypeStruct((B, V), jnp.bfloat16),
            jax.ShapeDtypeStruct((B, 1), jnp.float32),
        ],
    )(sp, sl, eosmask, gmat)


def _tc_combine(present, tsq):
    B, V = tsq.shape
    inv_v = 1.0 / V

    def body(pr_ref, tq_ref, loss_ref):
        pr = pr_ref[...]
        skip = pr.min(axis=0, keepdims=True)
        mask = pr * (1.0 - skip)
        tq = tq_ref[...].astype(jnp.float32)
        loss_ref[...] = (mask * tq).sum(axis=1, keepdims=True) * inv_v

    return pl.pallas_call(
        body,
        out_shape=jax.ShapeDtypeStruct((B, 1), jnp.float32),
    )(present, tsq)


def kernel(sentences_logits, visual_features, text_features, semantic_prior,
           semantic_prior_logits, grounding_signal, eos_idx):
    B, L, V = sentences_logits.shape
    ntf = text_features.shape[1]

    NW = _NUM_SC_CORES * _NUM_SC_SUBCORES
    RP = B // NW
    gs = grounding_signal.reshape(B, L)
    gs = jnp.clip(gs, 0, ntf - 1)
    local_row = (jnp.arange(B, dtype=jnp.int32) % RP) * ntf
    flat_idx = (gs + local_row[:, None]).reshape(-1)

    present = _sc_present(flat_idx, B, ntf, L).reshape(B, ntf)

    eosmask = (lax.broadcasted_iota(jnp.int32, (1, 1, V), 2)
               == eos_idx).astype(jnp.float32)
    sp_flat = semantic_prior.reshape(B, -1)
    NVF = semantic_prior.shape[2]
    VN = sp_flat.shape[1]
    gmat = (jnp.arange(VN, dtype=jnp.int32)[:, None] // NVF
            == jnp.arange(V, dtype=jnp.int32)[None, :]).astype(jnp.bfloat16)
    entropy, tsq, sle = _tc_dense(sp_flat, sentences_logits, eosmask, gmat,
                                  BB=64)

    sentences_loss = _tc_combine(present, tsq)

    loss = jnp.zeros((B, ntf), jnp.float32)
    return (loss, sentences_loss.reshape(B), entropy, sle.reshape(B))


# speedup vs baseline: 1.0841x; 1.0841x over previous
import dataclasses
import functools

import jax
import jax.numpy as jnp
from jax import lax
from jax.experimental import pallas as pl
from jax.experimental.pallas import tpu as pltpu
from jax.experimental.pallas import tpu_sc as plsc

_NUM_SC_CORES = 2
_NUM_SC_SUBCORES = 16
_SC_LANES = 16


def _sc_present(flat_idx, B, V, L):
    NW = _NUM_SC_CORES * _NUM_SC_SUBCORES
    RP = B // NW
    CH = RP * V
    NI = RP * L
    mesh = plsc.VectorSubcoreMesh(core_axis_name="c", subcore_axis_name="s")
    cp = pltpu.CompilerParams()
    if "needs_layout_passes" in pltpu.CompilerParams.__dataclass_fields__:
        cp = dataclasses.replace(cp, needs_layout_passes=False)

    @functools.partial(
        pl.kernel,
        out_type=jax.ShapeDtypeStruct((B * V,), jnp.float32),
        mesh=mesh,
        compiler_params=cp,
        scratch_types=[
            pltpu.VMEM((CH,), jnp.float32),
            pltpu.VMEM((NI,), jnp.int32),
        ],
    )
    def k(idx_hbm, out_hbm, buf, idxv):
        wid = lax.axis_index("s") * _NUM_SC_CORES + lax.axis_index("c")
        zeros = jnp.zeros((_SC_LANES,), jnp.float32)
        ones = jnp.ones((_SC_LANES,), jnp.float32)

        pltpu.sync_copy(idx_hbm.at[pl.ds(wid * NI, NI)], idxv)

        @pl.loop(0, CH, step=_SC_LANES)
        def _(j):
            buf[pl.ds(j, _SC_LANES)] = zeros

        @pl.loop(0, NI, step=_SC_LANES)
        def _(j):
            plsc.store_scatter(buf, [idxv[pl.ds(j, _SC_LANES)]], ones)

        pltpu.sync_copy(buf, out_hbm.at[pl.ds(wid * CH, CH)])

    return k(flat_idx)


def _tc_dense(sp, sl, eosmask, BB):
    B, L, V = sl.shape
    VN = sp.shape[1]
    NVF = VN // V

    def body(sp_ref, sl_ref, em_ref, ent_ref, tsq_ref, sle_ref, g_ref):
        @pl.when(pl.program_id(0) == 0)
        def _():
            r = lax.broadcasted_iota(jnp.int32, (VN, V), 0)
            c = lax.broadcasted_iota(jnp.int32, (VN, V), 1)
            g_ref[...] = ((r // NVF) == c).astype(jnp.bfloat16)

        p0 = sp_ref[...]
        t = (p0 * jnp.log(p0)).astype(jnp.bfloat16)
        ent_ref[...] = -lax.dot_general(
            t, g_ref[...], (((1,), (0,)), ((), ())),
            preferred_element_type=jnp.float32)
        x = sl_ref[...]
        mn = x.min(axis=2, keepdims=True)
        xs = jnp.where(em_ref[...] != 0, mn, x)
        m = xs.max(axis=1)
        mx = m.max(axis=1, keepdims=True)
        z = m - mx
        e = jnp.exp(z)
        s = e.sum(axis=1, keepdims=True)
        p = e / s
        logp = z - jnp.log(s)
        sle_ref[...] = -(p * logp).sum(axis=1, keepdims=True)
        t = 1.0 - p
        tsq_ref[...] = (t * t).astype(jnp.bfloat16)

    return pl.pallas_call(
        body,
        grid=(B // BB,),
        in_specs=[
            pl.BlockSpec((BB, VN), lambda i: (i, 0)),
            pl.BlockSpec((BB, L, V), lambda i: (i, 0, 0)),
            pl.BlockSpec((1, 1, V), lambda i: (0, 0, 0)),
        ],
        out_specs=[
            pl.BlockSpec((BB, V), lambda i: (i, 0)),
            pl.BlockSpec((BB, V), lambda i: (i, 0)),
            pl.BlockSpec((BB, 1), lambda i: (i, 0)),
        ],
        out_shape=[
            jax.ShapeDtypeStruct((B, V), jnp.float32),
            jax.ShapeDtypeStruct((B, V), jnp.bfloat16),
            jax.ShapeDtypeStruct((B, 1), jnp.float32),
        ],
        scratch_shapes=[pltpu.VMEM((VN, V), jnp.bfloat16)],
    )(sp, sl, eosmask)


def _tc_combine(present, tsq):
    B, V = tsq.shape
    inv_v = 1.0 / V

    def body(pr_ref, tq_ref, loss_ref):
        pr = pr_ref[...]
        skip = pr.min(axis=0, keepdims=True)
        mask = pr * (1.0 - skip)
        tq = tq_ref[...].astype(jnp.float32)
        loss_ref[...] = (mask * tq).sum(axis=1, keepdims=True) * inv_v

    return pl.pallas_call(
        body,
        out_shape=jax.ShapeDtypeStruct((B, 1), jnp.float32),
    )(present, tsq)


def kernel(sentences_logits, visual_features, text_features, semantic_prior,
           semantic_prior_logits, grounding_signal, eos_idx):
    B, L, V = sentences_logits.shape
    ntf = text_features.shape[1]

    NW = _NUM_SC_CORES * _NUM_SC_SUBCORES
    RP = B // NW
    gs = grounding_signal.reshape(B, L)
    gs = jnp.clip(gs, 0, ntf - 1)
    local_row = (jnp.arange(B, dtype=jnp.int32) % RP) * ntf
    flat_idx = (gs + local_row[:, None]).reshape(-1)

    present = _sc_present(flat_idx, B, ntf, L).reshape(B, ntf)

    eosmask = (lax.broadcasted_iota(jnp.int32, (1, 1, V), 2)
               == eos_idx).astype(jnp.float32)
    sp_flat = semantic_prior.reshape(B, -1)
    entropy, tsq, sle = _tc_dense(sp_flat, sentences_logits, eosmask,
                                  BB=128)

    sentences_loss = _tc_combine(present, tsq)

    loss = jnp.zeros((B, ntf), jnp.float32)
    return (loss, sentences_loss.reshape(B), entropy, sle.reshape(B))


# speedup vs baseline: 1.0888x; 1.0043x over previous
import dataclasses
import functools

import jax
import jax.numpy as jnp
from jax import lax
from jax.experimental import pallas as pl
from jax.experimental.pallas import tpu as pltpu
from jax.experimental.pallas import tpu_sc as plsc

_NUM_SC_CORES = 2
_NUM_SC_SUBCORES = 16
_SC_LANES = 16


def _sc_present(flat_idx, B, V, L):
    NW = _NUM_SC_CORES * _NUM_SC_SUBCORES
    RP = B // NW
    CH = RP * V
    NI = RP * L
    mesh = plsc.VectorSubcoreMesh(core_axis_name="c", subcore_axis_name="s")
    cp = pltpu.CompilerParams()
    if "needs_layout_passes" in pltpu.CompilerParams.__dataclass_fields__:
        cp = dataclasses.replace(cp, needs_layout_passes=False)

    @functools.partial(
        pl.kernel,
        out_type=jax.ShapeDtypeStruct((B * V,), jnp.float32),
        mesh=mesh,
        compiler_params=cp,
        scratch_types=[
            pltpu.VMEM((CH,), jnp.float32),
            pltpu.VMEM((NI,), jnp.int32),
        ],
    )
    def k(idx_hbm, out_hbm, buf, idxv):
        wid = lax.axis_index("s") * _NUM_SC_CORES + lax.axis_index("c")
        zeros = jnp.zeros((_SC_LANES,), jnp.float32)
        ones = jnp.ones((_SC_LANES,), jnp.float32)

        pltpu.sync_copy(idx_hbm.at[pl.ds(wid * NI, NI)], idxv)

        @pl.loop(0, CH, step=_SC_LANES)
        def _(j):
            buf[pl.ds(j, _SC_LANES)] = zeros

        @pl.loop(0, NI, step=_SC_LANES)
        def _(j):
            plsc.store_scatter(buf, [idxv[pl.ds(j, _SC_LANES)]], ones)

        pltpu.sync_copy(buf, out_hbm.at[pl.ds(wid * CH, CH)])

    return k(flat_idx)


def _tc_dense(sp, sl, eosmask, BB):
    B, L, V = sl.shape
    VN = sp.shape[1]
    NVF = VN // V

    def body(sp_ref, sl_ref, em_ref, ent_ref, tsq_ref, sle_ref, g_ref):
        @pl.when(pl.program_id(0) == 0)
        def _():
            r = lax.broadcasted_iota(jnp.int32, (VN, V), 0)
            c = lax.broadcasted_iota(jnp.int32, (VN, V), 1)
            g_ref[...] = ((r // NVF) == c).astype(jnp.bfloat16)

        p0 = sp_ref[...]
        t = (p0 * jnp.log(p0)).astype(jnp.bfloat16)
        ent_ref[...] = -lax.dot_general(
            t, g_ref[...], (((1,), (0,)), ((), ())),
            preferred_element_type=jnp.float32)
        x = sl_ref[...]
        mn = x.min(axis=2, keepdims=True)
        xs = jnp.where(em_ref[...] != 0, mn, x)
        m = xs.max(axis=1)
        mx = m.max(axis=1, keepdims=True)
        z = m - mx
        e = jnp.exp(z)
        s = e.sum(axis=1, keepdims=True)
        p = e / s
        logp = z - jnp.log(s)
        sle_ref[...] = -(p * logp).sum(axis=1, keepdims=True)
        t = 1.0 - p
        tsq_ref[...] = (t * t).astype(jnp.bfloat16)

    return pl.pallas_call(
        body,
        grid=(B // BB,),
        in_specs=[
            pl.BlockSpec((BB, VN), lambda i: (i, 0)),
            pl.BlockSpec((BB, L, V), lambda i: (i, 0, 0)),
            pl.BlockSpec((1, 1, V), lambda i: (0, 0, 0)),
        ],
        out_specs=[
            pl.BlockSpec((BB, V), lambda i: (i, 0)),
            pl.BlockSpec((BB, V), lambda i: (i, 0)),
            pl.BlockSpec((BB, 1), lambda i: (i, 0)),
        ],
        out_shape=[
            jax.ShapeDtypeStruct((B, V), jnp.float32),
            jax.ShapeDtypeStruct((B, V), jnp.bfloat16),
            jax.ShapeDtypeStruct((B, 1), jnp.float32),
        ],
        scratch_shapes=[pltpu.VMEM((VN, V), jnp.bfloat16)],
    )(sp, sl, eosmask)


def _tc_combine(present, tsq):
    B, V = tsq.shape
    inv_v = 1.0 / V

    def body(pr_ref, tq_ref, loss_ref):
        pr = pr_ref[...]
        skip = pr.min(axis=0, keepdims=True)
        mask = pr * (1.0 - skip)
        tq = tq_ref[...].astype(jnp.float32)
        loss_ref[...] = (mask * tq).sum(axis=1, keepdims=True) * inv_v

    return pl.pallas_call(
        body,
        out_shape=jax.ShapeDtypeStruct((B, 1), jnp.float32),
    )(present, tsq)


def kernel(sentences_logits, visual_features, text_features, semantic_prior,
           semantic_prior_logits, grounding_signal, eos_idx):
    B, L, V = sentences_logits.shape
    ntf = text_features.shape[1]

    NW = _NUM_SC_CORES * _NUM_SC_SUBCORES
    RP = B // NW
    gs = grounding_signal.reshape(B, L)
    gs = jnp.clip(gs, 0, ntf - 1)
    local_row = (jnp.arange(B, dtype=jnp.int32) % RP) * ntf
    flat_idx = (gs + local_row[:, None]).reshape(-1)

    present = _sc_present(flat_idx, B, ntf, L).reshape(B, ntf)

    eosmask = (lax.broadcasted_iota(jnp.int32, (1, 1, V), 2)
               == eos_idx).astype(jnp.float32)
    sp_flat = semantic_prior.reshape(B, -1)
    entropy, tsq, sle = _tc_dense(sp_flat, sentences_logits, eosmask,
                                  BB=256)

    sentences_loss = _tc_combine(present, tsq)

    loss = jnp.zeros((B, ntf), jnp.float32)
    return (loss, sentences_loss.reshape(B), entropy, sle.reshape(B))
